# Initial kernel scaffold; baseline (speedup 1.0000x reference)
import jax
import jax.numpy as jnp
from jax.experimental import pallas as pl


def kernel(features, coords, edge_index, q_w, q_b, k_w, k_b, v_w, v_b, aw_w, aw_b, out_w, out_b):
    raise NotImplementedError("write your pallas kernel here")


# speedup vs baseline: 19.0088x; 19.0088x over previous
import functools
import math

import jax
import jax.numpy as jnp
from jax import lax
from jax.experimental import pallas as pl
from jax.experimental.pallas import tpu as pltpu
from jax.experimental.pallas import tpu_sc as plsc

N_NODES = 10000
FEAT = 128
HEADS = 8
HDIM = 16
QC_W = 2 * FEAT
KVC_W = 3 * FEAT
NC = 2
NS = 16
NW = NC * NS
CHUNK = 40


def _pre_body(f_ref, cpad_ref, wq_ref, bq_ref, wk_ref, bk_ref, wv_ref, bv_ref,
              qc_ref, kvc_ref):
    f = f_ref[...]
    n = f.shape[0]
    cpad = cpad_ref[...]
    hp = lax.Precision.HIGHEST
    q = jnp.dot(f, wq_ref[...], preferred_element_type=jnp.float32,
                precision=hp) + bq_ref[...]
    k = jnp.dot(f, wk_ref[...], preferred_element_type=jnp.float32,
                precision=hp) + bk_ref[...]
    v = jnp.dot(f, wv_ref[...], preferred_element_type=jnp.float32,
                precision=hp) + bv_ref[...]
    z112 = jnp.zeros((n, 112), jnp.float32)
    qc_ref[...] = jnp.concatenate([q, cpad, z112], axis=1)
    kvc_ref[...] = jnp.concatenate([k, v, cpad, z112], axis=1)


def _post_body(acc_ref, den_ref, w_ref, b_ref, o_ref):
    n = o_ref.shape[0]
    hp = lax.Precision.HIGHEST
    dsum = jnp.sum(den_ref[...], axis=0)
    lane = lax.iota(jnp.int32, FEAT)
    recip = jnp.where(lane < 8, 1.0 / jnp.maximum(dsum, 1e-30), 0.0)
    a_io = lax.broadcasted_iota(jnp.int32, (FEAT, FEAT), 0)
    b_io = lax.broadcasted_iota(jnp.int32, (FEAT, FEAT), 1)
    sel = (a_io == b_io // HDIM).astype(jnp.float32)
    scale = jnp.dot(recip.reshape(1, FEAT), sel,
                    preferred_element_type=jnp.float32, precision=hp)
    a = (acc_ref[0, :n] + acc_ref[1, :n]) * scale
    o_ref[...] = jnp.dot(a, w_ref[...], preferred_element_type=jnp.float32,
                         precision=hp) + b_ref[...]


def _exp_sw(x):
    log2e = jnp.float32(1.4426950408889634)
    ln2 = jnp.float32(0.6931471805599453)
    z = x * log2e
    zr = jnp.where(z >= 0.0, z + 0.5, z - 0.5)
    ni = zr.astype(jnp.int32)
    nf = ni.astype(jnp.float32)
    r = x - nf * ln2
    p = jnp.float32(1.0 / 120.0)
    p = p * r + jnp.float32(1.0 / 24.0)
    p = p * r + jnp.float32(1.0 / 6.0)
    p = p * r + jnp.float32(0.5)
    p = p * r + jnp.float32(1.0)
    p = p * r + jnp.float32(1.0)
    scale = plsc.bitcast(lax.shift_left(ni + 127, 23), jnp.float32)
    return p * scale


def _sc_body(row_hbm, col_hbm, qc_hbm, kvc_hbm, aww_hbm, awb_hbm,
             acc_hbm, den_hbm,
             ridx_v, cidx_v, qc_v, kvc_v, msg_v, e_v, aww_v, awb_v,
             acc_sh, gsem):
    cid = lax.axis_index("c")
    sid = lax.axis_index("s")
    wid = sid * NC + cid

    pltpu.sync_copy(aww_hbm, aww_v)
    pltpu.sync_copy(awb_hbm, awb_v)
    aww = aww_v[...]
    awb = awb_v[...]

    zero16 = jnp.zeros((16,), jnp.float32)
    lanei = lax.iota(jnp.int32, 16)
    maskv = jnp.where(lanei < 8, 1.0, 0.0)
    onehots = [(lanei == h).astype(jnp.float32) for h in range(HEADS)]

    for j in range(8):
        e_v[pl.ds(j * 16, 16)] = zero16

    def _zrow(i, x):
        for j in range(8):
            msg_v[i, pl.ds(j * 16, 16)] = zero16
        return x
    lax.fori_loop(0, CHUNK, _zrow, 0)
    rows_per_tile = acc_sh.shape[0] // NS
    zbase = sid * rows_per_tile
    off = 0
    while off < rows_per_tile:
        sz = min(CHUNK, rows_per_tile - off)
        pltpu.sync_copy(msg_v.at[pl.ds(0, sz)],
                        acc_sh.at[pl.ds(zbase + off, sz)])
        off += sz
    plsc.subcore_barrier()

    e_per_tile = row_hbm.shape[0] // NW
    n_chunks = e_per_tile // CHUNK
    ebase = wid * e_per_tile

    def _chunk(g, den):
        eb = ebase + g * CHUNK
        pltpu.sync_copy(row_hbm.at[pl.ds(eb, CHUNK)], ridx_v)
        pltpu.sync_copy(col_hbm.at[pl.ds(eb, CHUNK)], cidx_v)
        cp1 = pltpu.async_copy(qc_hbm.at[ridx_v], qc_v, gsem)
        cp2 = pltpu.async_copy(kvc_hbm.at[cidx_v], kvc_v, gsem)
        cp1.wait()
        cp2.wait()

        def _edge(i, den_in):
            cr = qc_v[i, pl.ds(FEAT, 16)]
            cc = kvc_v[i, pl.ds(2 * FEAT, 16)]
            df = cr - cc
            d2s = jnp.sum(df * df) + 1e-12
            d2 = lax.broadcast(d2s, (16,))
            bi = plsc.bitcast(d2, jnp.int32)
            bi = jnp.int32(0x5F3759DF) - lax.shift_right_logical(bi, 1)
            y = plsc.bitcast(bi, jnp.float32)
            for _ in range(3):
                y = y * (1.5 - 0.5 * d2 * y * y)
            dv = d2 * y
            t = dv * aww + awb
            u = 1.0 + _exp_sw(-t)
            gb = plsc.bitcast(jnp.int32(0x7EF311C3) - plsc.bitcast(u, jnp.int32),
                              jnp.float32)
            for _ in range(3):
                gb = gb * (2.0 - u * gb)
            sv = zero16
            for h in range(HEADS):
                qh = qc_v[i, pl.ds(h * 16, 16)]
                kh = kvc_v[i, pl.ds(h * 16, 16)]
                s_h = jnp.sum(qh * kh)
                sv = jnp.where(lanei == h, lax.broadcast(s_h, (16,)), sv)
            ev = _exp_sw(sv * gb) * maskv
            for h in range(HEADS):
                e_h = jnp.sum(ev * onehots[h])
                msg_v[i, pl.ds(h * 16, 16)] = (
                    lax.broadcast(e_h, (16,)) * kvc_v[i, pl.ds(FEAT + h * 16, 16)])
            return den_in + ev

        den = lax.fori_loop(0, CHUNK, _edge, den)
        pltpu.sync_copy(msg_v, acc_sh.at[ridx_v], add=True)
        return den

    den = lax.fori_loop(0, n_chunks, _chunk, jnp.zeros((16,), jnp.float32))

    e_v[pl.ds(0, 16)] = den
    pltpu.sync_copy(e_v, den_hbm.at[wid])

    plsc.subcore_barrier()
    off = 0
    while off < rows_per_tile:
        sz = min(CHUNK, rows_per_tile - off)
        pltpu.sync_copy(acc_sh.at[pl.ds(zbase + off, sz)], msg_v.at[pl.ds(0, sz)])
        pltpu.sync_copy(msg_v.at[pl.ds(0, sz)],
                        acc_hbm.at[cid, pl.ds(zbase + off, sz)])
        off += sz


def kernel(features, coords, edge_index, q_w, q_b, k_w, k_b, v_w, v_b,
           aw_w, aw_b, out_w, out_b):
    n = features.shape[0]
    scale = 1.0 / math.sqrt(HDIM)
    wq = q_w.T * scale
    bq = (q_b * scale).reshape(1, FEAT)
    wk = k_w.T
    bk = k_b.reshape(1, FEAT)
    wv = v_w.T
    bv = v_b.reshape(1, FEAT)
    cpad = jnp.pad(coords.astype(jnp.float32), ((0, 0), (0, 13)))

    rows_b = 1000
    qc, kvc = pl.pallas_call(
        _pre_body,
        grid=(n // rows_b,),
        in_specs=[
            pl.BlockSpec((rows_b, FEAT), lambda i: (i, 0)),
            pl.BlockSpec((rows_b, 16), lambda i: (i, 0)),
            pl.BlockSpec((FEAT, FEAT), lambda i: (0, 0)),
            pl.BlockSpec((1, FEAT), lambda i: (0, 0)),
            pl.BlockSpec((FEAT, FEAT), lambda i: (0, 0)),
            pl.BlockSpec((1, FEAT), lambda i: (0, 0)),
            pl.BlockSpec((FEAT, FEAT), lambda i: (0, 0)),
            pl.BlockSpec((1, FEAT), lambda i: (0, 0)),
        ],
        out_specs=[
            pl.BlockSpec((rows_b, QC_W), lambda i: (i, 0)),
            pl.BlockSpec((rows_b, KVC_W), lambda i: (i, 0)),
        ],
        out_shape=[
            jax.ShapeDtypeStruct((n, QC_W), jnp.float32),
            jax.ShapeDtypeStruct((n, KVC_W), jnp.float32),
        ],
    )(features, cpad, wq, bq, wk, bk, wv, bv)

    row = edge_index[0].astype(jnp.int32)
    col = edge_index[1].astype(jnp.int32)
    aww = jnp.pad(aw_w.reshape(-1).astype(jnp.float32), (0, 16 - HEADS))
    awb = jnp.pad(aw_b.astype(jnp.float32), (0, 16 - HEADS))

    npad = ((n + 127) // 128) * 128
    mesh = plsc.VectorSubcoreMesh(core_axis_name="c", subcore_axis_name="s")
    sc_fn = functools.partial(
        pl.kernel,
        out_type=[
            jax.ShapeDtypeStruct((NC, npad, FEAT), jnp.float32),
            jax.ShapeDtypeStruct((NW, FEAT), jnp.float32),
        ],
        mesh=mesh,
        scratch_types=[
            pltpu.VMEM((CHUNK,), jnp.int32),
            pltpu.VMEM((CHUNK,), jnp.int32),
            pltpu.VMEM((CHUNK, QC_W), jnp.float32),
            pltpu.VMEM((CHUNK, KVC_W), jnp.float32),
            pltpu.VMEM((CHUNK, FEAT), jnp.float32),
            pltpu.VMEM((FEAT,), jnp.float32),
            pltpu.VMEM((16,), jnp.float32),
            pltpu.VMEM((16,), jnp.float32),
            pltpu.VMEM_SHARED((npad, FEAT), jnp.float32),
            pltpu.SemaphoreType.DMA,
        ],
        compiler_params=pltpu.CompilerParams(use_tc_tiling_on_sc=False,
                                             needs_layout_passes=False),
    )(_sc_body)
    acc, dens = sc_fn(row, col, qc, kvc, aww, awb)

    out = pl.pallas_call(
        _post_body,
        out_shape=jax.ShapeDtypeStruct((n, FEAT), jnp.float32),
    )(acc, dens, out_w.T, out_b.reshape(1, FEAT))
    return out


# speedup vs baseline: 27.2694x; 1.4346x over previous
import functools
import math

import jax
import jax.numpy as jnp
from jax import lax
from jax.experimental import pallas as pl
from jax.experimental.pallas import tpu as pltpu
from jax.experimental.pallas import tpu_sc as plsc

N_NODES = 10000
FEAT = 128
HEADS = 8
HDIM = 16
QC_W = 2 * FEAT
KVC_W = 3 * FEAT
NC = 2
NS = 16
NW = NC * NS
CHUNK = 16


def _pre_body(f_ref, cpad_ref, wq_ref, bq_ref, wk_ref, bk_ref, wv_ref, bv_ref,
              qc_ref, kvc_ref):
    f = f_ref[...]
    n = f.shape[0]
    cpad = cpad_ref[...]
    hp = lax.Precision.HIGHEST
    q = jnp.dot(f, wq_ref[...], preferred_element_type=jnp.float32,
                precision=hp) + bq_ref[...]
    k = jnp.dot(f, wk_ref[...], preferred_element_type=jnp.float32,
                precision=hp) + bk_ref[...]
    v = jnp.dot(f, wv_ref[...], preferred_element_type=jnp.float32,
                precision=hp) + bv_ref[...]
    z112 = jnp.zeros((n, 112), jnp.float32)
    qc_ref[...] = jnp.concatenate([q, cpad, z112], axis=1)
    kvc_ref[...] = jnp.concatenate([k, v, cpad, z112], axis=1)


def _post_body(acc_ref, den_ref, w_ref, b_ref, o_ref):
    n = o_ref.shape[0]
    hp = lax.Precision.HIGHEST
    dsum = jnp.sum(den_ref[...], axis=0)
    lane = lax.iota(jnp.int32, FEAT)
    recip = jnp.where(lane < 8, 1.0 / jnp.maximum(dsum, 1e-30), 0.0)
    a_io = lax.broadcasted_iota(jnp.int32, (FEAT, FEAT), 0)
    b_io = lax.broadcasted_iota(jnp.int32, (FEAT, FEAT), 1)
    sel = (a_io == b_io // HDIM).astype(jnp.float32)
    scale = jnp.dot(recip.reshape(1, FEAT), sel,
                    preferred_element_type=jnp.float32, precision=hp)
    a = (acc_ref[0, :n] + acc_ref[1, :n]) * scale
    o_ref[...] = jnp.dot(a, w_ref[...], preferred_element_type=jnp.float32,
                         precision=hp) + b_ref[...]


def _exp_sw(x):
    log2e = jnp.float32(1.4426950408889634)
    ln2 = jnp.float32(0.6931471805599453)
    z = x * log2e
    zr = jnp.where(z >= 0.0, z + 0.5, z - 0.5)
    ni = zr.astype(jnp.int32)
    nf = ni.astype(jnp.float32)
    r = x - nf * ln2
    p = jnp.float32(1.0 / 120.0)
    p = p * r + jnp.float32(1.0 / 24.0)
    p = p * r + jnp.float32(1.0 / 6.0)
    p = p * r + jnp.float32(0.5)
    p = p * r + jnp.float32(1.0)
    p = p * r + jnp.float32(1.0)
    scale = plsc.bitcast(lax.shift_left(ni + 127, 23), jnp.float32)
    return p * scale


def _sc_body(row_hbm, col_hbm, qc_hbm, kvc_hbm, aww_hbm, awb_hbm,
             acc_hbm, den_hbm,
             ridx_all, cidx_all, qc_v, kvc_v, msg_v, sidx_v, e_v,
             aww_v, awb_v, acc_sh, gsem0, gsem1, ssem0, ssem1):
    cid = lax.axis_index("c")
    sid = lax.axis_index("s")
    wid = sid * NC + cid
    gsems = (gsem0, gsem1)
    ssems = (ssem0, ssem1)

    pltpu.sync_copy(aww_hbm, aww_v)
    pltpu.sync_copy(awb_hbm, awb_v)
    aww = aww_v[...]
    awb = awb_v[...]

    zero16 = jnp.zeros((16,), jnp.float32)
    lanei = lax.iota(jnp.int32, 16)
    maskv = jnp.where(lanei < 8, 1.0, 0.0)
    onehots = [(lanei == h).astype(jnp.float32) for h in range(HEADS)]

    for j in range(8):
        e_v[pl.ds(j * 16, 16)] = zero16

    def _zrow(i, x):
        for j in range(8):
            msg_v[i, pl.ds(j * 16, 16)] = zero16
        return x
    lax.fori_loop(0, 2 * CHUNK, _zrow, 0)
    rows_per_tile = acc_sh.shape[0] // NS
    zbase = sid * rows_per_tile
    off = 0
    while off < rows_per_tile:
        sz = min(2 * CHUNK, rows_per_tile - off)
        pltpu.sync_copy(msg_v.at[pl.ds(0, sz)],
                        acc_sh.at[pl.ds(zbase + off, sz)])
        off += sz
    plsc.subcore_barrier()

    e_per_tile = row_hbm.shape[0] // NW
    n_chunks = e_per_tile // CHUNK
    ebase = wid * e_per_tile
    pltpu.sync_copy(row_hbm.at[pl.ds(ebase, e_per_tile)], ridx_all)
    pltpu.sync_copy(col_hbm.at[pl.ds(ebase, e_per_tile)], cidx_all)

    def _issue_gathers(g, b):
        pltpu.async_copy(qc_hbm.at[ridx_all.at[pl.ds(g * CHUNK, CHUNK)]],
                         qc_v.at[pl.ds(b * CHUNK, CHUNK)], gsems[b])
        pltpu.async_copy(kvc_hbm.at[cidx_all.at[pl.ds(g * CHUNK, CHUNK)]],
                         kvc_v.at[pl.ds(b * CHUNK, CHUNK)], gsems[b])

    def _wait_gathers(g, b):
        pltpu.make_async_copy(qc_hbm.at[ridx_all.at[pl.ds(g * CHUNK, CHUNK)]],
                              qc_v.at[pl.ds(b * CHUNK, CHUNK)], gsems[b]).wait()
        pltpu.make_async_copy(kvc_hbm.at[cidx_all.at[pl.ds(g * CHUNK, CHUNK)]],
                              kvc_v.at[pl.ds(b * CHUNK, CHUNK)], gsems[b]).wait()

    def _issue_scatter(b):
        pltpu.async_copy(msg_v.at[pl.ds(b * CHUNK, CHUNK)],
                         acc_sh.at[sidx_v.at[b]], ssems[b], add=True)

    def _wait_scatter(b):
        pltpu.make_async_copy(msg_v.at[pl.ds(b * CHUNK, CHUNK)],
                              acc_sh.at[sidx_v.at[b]], ssems[b]).wait()

    def _compute(g, b, den):
        bo = b * CHUNK
        sidx_v[b, pl.ds(0, 16)] = ridx_all[pl.ds(g * CHUNK, 16)]

        def _edge(i, den_in):
            r = bo + i
            cr = qc_v[r, pl.ds(FEAT, 16)]
            cc = kvc_v[r, pl.ds(2 * FEAT, 16)]
            df = cr - cc
            d2s = jnp.sum(df * df) + 1e-12
            d2 = lax.broadcast(d2s, (16,))
            bi = plsc.bitcast(d2, jnp.int32)
            bi = jnp.int32(0x5F3759DF) - lax.shift_right_logical(bi, 1)
            y = plsc.bitcast(bi, jnp.float32)
            for _ in range(3):
                y = y * (1.5 - 0.5 * d2 * y * y)
            dv = d2 * y
            t = dv * aww + awb
            u = 1.0 + _exp_sw(-t)
            gb = plsc.bitcast(jnp.int32(0x7EF311C3) - plsc.bitcast(u, jnp.int32),
                              jnp.float32)
            for _ in range(3):
                gb = gb * (2.0 - u * gb)
            sv = zero16
            for h in range(HEADS):
                qh = qc_v[r, pl.ds(h * 16, 16)]
                kh = kvc_v[r, pl.ds(h * 16, 16)]
                s_h = jnp.sum(qh * kh)
                sv = jnp.where(lanei == h, lax.broadcast(s_h, (16,)), sv)
            ev = _exp_sw(sv * gb) * maskv
            for h in range(HEADS):
                e_h = jnp.sum(ev * onehots[h])
                msg_v[r, pl.ds(h * 16, 16)] = (
                    lax.broadcast(e_h, (16,)) * kvc_v[r, pl.ds(FEAT + h * 16, 16)])
            return den_in + ev

        return lax.fori_loop(0, CHUNK, _edge, den)

    _issue_gathers(0, 0)
    _wait_gathers(0, 0)
    _issue_gathers(1, 1)
    den = _compute(0, 0, jnp.zeros((16,), jnp.float32))
    _issue_scatter(0)

    def _pair(go, den):
        ga = 2 * go + 1
        _wait_gathers(ga, 1)
        _issue_gathers(ga + 1, 0)

        @pl.when(go > 0)
        def _():
            _wait_scatter(1)
        den = _compute(ga, 1, den)
        _issue_scatter(1)

        gb2 = 2 * go + 2
        _wait_gathers(gb2, 0)

        @pl.when(go < (n_chunks - 3) // 2)
        def _():
            _issue_gathers(gb2 + 1, 1)
        _wait_scatter(0)
        den = _compute(gb2, 0, den)
        _issue_scatter(0)
        return den

    den = lax.fori_loop(0, (n_chunks - 1) // 2, _pair, den)
    _wait_scatter(1)
    _wait_scatter(0)

    e_v[pl.ds(0, 16)] = den
    pltpu.sync_copy(e_v, den_hbm.at[wid])

    plsc.subcore_barrier()
    off = 0
    while off < rows_per_tile:
        sz = min(2 * CHUNK, rows_per_tile - off)
        pltpu.sync_copy(acc_sh.at[pl.ds(zbase + off, sz)], msg_v.at[pl.ds(0, sz)])
        pltpu.sync_copy(msg_v.at[pl.ds(0, sz)],
                        acc_hbm.at[cid, pl.ds(zbase + off, sz)])
        off += sz


def kernel(features, coords, edge_index, q_w, q_b, k_w, k_b, v_w, v_b,
           aw_w, aw_b, out_w, out_b):
    n = features.shape[0]
    scale = 1.0 / math.sqrt(HDIM)
    wq = q_w.T * scale
    bq = (q_b * scale).reshape(1, FEAT)
    wk = k_w.T
    bk = k_b.reshape(1, FEAT)
    wv = v_w.T
    bv = v_b.reshape(1, FEAT)
    cpad = jnp.pad(coords.astype(jnp.float32), ((0, 0), (0, 13)))

    rows_b = 1000
    qc, kvc = pl.pallas_call(
        _pre_body,
        grid=(n // rows_b,),
        in_specs=[
            pl.BlockSpec((rows_b, FEAT), lambda i: (i, 0)),
            pl.BlockSpec((rows_b, 16), lambda i: (i, 0)),
            pl.BlockSpec((FEAT, FEAT), lambda i: (0, 0)),
            pl.BlockSpec((1, FEAT), lambda i: (0, 0)),
            pl.BlockSpec((FEAT, FEAT), lambda i: (0, 0)),
            pl.BlockSpec((1, FEAT), lambda i: (0, 0)),
            pl.BlockSpec((FEAT, FEAT), lambda i: (0, 0)),
            pl.BlockSpec((1, FEAT), lambda i: (0, 0)),
        ],
        out_specs=[
            pl.BlockSpec((rows_b, QC_W), lambda i: (i, 0)),
            pl.BlockSpec((rows_b, KVC_W), lambda i: (i, 0)),
        ],
        out_shape=[
            jax.ShapeDtypeStruct((n, QC_W), jnp.float32),
            jax.ShapeDtypeStruct((n, KVC_W), jnp.float32),
        ],
    )(features, cpad, wq, bq, wk, bk, wv, bv)

    row = edge_index[0].astype(jnp.int32)
    col = edge_index[1].astype(jnp.int32)
    aww = jnp.pad(aw_w.reshape(-1).astype(jnp.float32), (0, 16 - HEADS))
    awb = jnp.pad(aw_b.astype(jnp.float32), (0, 16 - HEADS))

    npad = ((n + 127) // 128) * 128
    e_per_tile = edge_index.shape[1] // NW
    mesh = plsc.VectorSubcoreMesh(core_axis_name="c", subcore_axis_name="s")
    sc_fn = functools.partial(
        pl.kernel,
        out_type=[
            jax.ShapeDtypeStruct((NC, npad, FEAT), jnp.float32),
            jax.ShapeDtypeStruct((NW, FEAT), jnp.float32),
        ],
        mesh=mesh,
        scratch_types=[
            pltpu.VMEM((e_per_tile,), jnp.int32),
            pltpu.VMEM((e_per_tile,), jnp.int32),
            pltpu.VMEM((2 * CHUNK, QC_W), jnp.float32),
            pltpu.VMEM((2 * CHUNK, KVC_W), jnp.float32),
            pltpu.VMEM((2 * CHUNK, FEAT), jnp.float32),
            pltpu.VMEM((2, 16), jnp.int32),
            pltpu.VMEM((FEAT,), jnp.float32),
            pltpu.VMEM((16,), jnp.float32),
            pltpu.VMEM((16,), jnp.float32),
            pltpu.VMEM_SHARED((npad, FEAT), jnp.float32),
            pltpu.SemaphoreType.DMA,
            pltpu.SemaphoreType.DMA,
            pltpu.SemaphoreType.DMA,
            pltpu.SemaphoreType.DMA,
        ],
        compiler_params=pltpu.CompilerParams(use_tc_tiling_on_sc=False,
                                             needs_layout_passes=False),
    )(_sc_body)
    acc, dens = sc_fn(row, col, qc, kvc, aww, awb)

    out = pl.pallas_call(
        _post_body,
        out_shape=jax.ShapeDtypeStruct((n, FEAT), jnp.float32),
    )(acc, dens, out_w.T, out_b.reshape(1, FEAT))
    return out
